# baseline (speedup 1.0000x reference)
import functools

import jax
import jax.numpy as jnp
from jax.experimental import pallas as pl
from jax.experimental.pallas import tpu as pltpu

_N_BINS = 15
_LANES = 16


def _ece_body(n_total, bounds_ref, logits_ref, labels_ref, ece_ref, acc_ref, hist_ref):
    pid = pl.program_id(0)
    nsteps = pl.num_programs(0)

    @pl.when(pid == 0)
    def _init():
        hist_ref[...] = jnp.zeros_like(hist_ref)

    x = logits_ref[...]
    lab = labels_ref[...]
    conf = jnp.max(x, axis=1, keepdims=True)
    col = jax.lax.broadcasted_iota(jnp.int32, x.shape, 1)
    pred = jnp.min(
        jnp.where(x == conf, col, jnp.int32(x.shape[1])), axis=1, keepdims=True
    )
    accv = (pred == lab).astype(jnp.float32)

    lowers = bounds_ref[0:1, :]
    uppers = bounds_ref[1:2, :]
    onehot = ((conf > lowers) & (conf <= uppers)).astype(jnp.float32)

    cnt = jnp.sum(onehot, axis=0, keepdims=True)
    sconf = jnp.sum(onehot * conf, axis=0, keepdims=True)
    sacc = jnp.sum(onehot * accv, axis=0, keepdims=True)
    hist_ref[...] += jnp.concatenate([cnt, sconf, sacc], axis=0)

    @pl.when(pid == nsteps - 1)
    def _finish():
        h = hist_ref[...]
        cntf = h[0:1, :]
        sc = h[1:2, :]
        sa = h[2:3, :]
        denom = jnp.maximum(cntf, 1.0)
        avg_conf = sc / denom
        avg_acc = sa / denom
        prop = cntf / jnp.float32(n_total)
        nonempty = cntf > 0.0
        ece_bins = jnp.where(nonempty, jnp.abs(avg_conf - avg_acc) * prop, 0.0)
        acc_bins = jnp.where(nonempty, avg_acc * prop, 0.0)
        ece_ref[...] = jnp.sum(ece_bins, axis=1, keepdims=True) * 100.0
        acc_ref[...] = jnp.sum(acc_bins, axis=1, keepdims=True) * 100.0


@jax.jit
def kernel(logits, labels):
    n, c = logits.shape
    block = 4096
    grid = n // block

    bounds = jnp.linspace(0.0, 1.0, _N_BINS + 1)
    lowers = jnp.concatenate([bounds[:-1], jnp.full((1,), 2.0, jnp.float32)])
    uppers = jnp.concatenate([bounds[1:], jnp.full((1,), 2.0, jnp.float32)])
    bounds2 = jnp.stack([lowers, uppers])

    labels2 = labels.astype(jnp.int32).reshape(n, 1)

    ece, acc = pl.pallas_call(
        functools.partial(_ece_body, n),
        grid=(grid,),
        in_specs=[
            pl.BlockSpec((2, _LANES), lambda i: (0, 0)),
            pl.BlockSpec((block, c), lambda i: (i, 0)),
            pl.BlockSpec((block, 1), lambda i: (i, 0)),
        ],
        out_specs=[
            pl.BlockSpec((1, 1), lambda i: (0, 0)),
            pl.BlockSpec((1, 1), lambda i: (0, 0)),
        ],
        out_shape=[
            jax.ShapeDtypeStruct((1, 1), jnp.float32),
            jax.ShapeDtypeStruct((1, 1), jnp.float32),
        ],
        scratch_shapes=[pltpu.VMEM((3, _LANES), jnp.float32)],
        compiler_params=pltpu.CompilerParams(
            dimension_semantics=("arbitrary",),
        ),
    )(bounds2, logits, labels2)
    return ece.reshape(1), acc.reshape(1)


# speedup vs baseline: 1.1945x; 1.1945x over previous
import functools

import jax
import jax.numpy as jnp
from jax.experimental import pallas as pl
from jax.experimental.pallas import tpu as pltpu

_N_BINS = 15
_LANES = 16


def _ece_body(n_total, bounds_ref, logits_ref, labels_ref, ece_ref, acc_ref, hist_ref):
    pid = pl.program_id(0)
    nsteps = pl.num_programs(0)

    @pl.when(pid == 0)
    def _init():
        hist_ref[...] = jnp.zeros_like(hist_ref)

    x = logits_ref[...]
    b, c = x.shape
    ident = (
        jax.lax.broadcasted_iota(jnp.int32, (c, c), 0)
        == jax.lax.broadcasted_iota(jnp.int32, (c, c), 1)
    ).astype(jnp.float32)
    xt = jax.lax.dot_general(
        ident, x, (((1,), (1,)), ((), ())), precision=jax.lax.Precision.HIGHEST
    )

    conf = jnp.max(xt, axis=0, keepdims=True)
    row = jax.lax.broadcasted_iota(jnp.int32, (c, b), 0)
    pred = jnp.min(
        jnp.where(xt == conf, row, jnp.int32(c)), axis=0, keepdims=True
    )
    lab = labels_ref[...].reshape(1, b)
    accv = (pred == lab).astype(jnp.float32)

    lowers = bounds_ref[0:1, :].reshape(_LANES, 1)
    uppers = bounds_ref[1:2, :].reshape(_LANES, 1)
    onehot = ((conf > lowers) & (conf <= uppers)).astype(jnp.float32)

    ones = jnp.ones((1, b), jnp.float32)
    stacked = jnp.concatenate([ones, conf, accv], axis=0)
    part = jax.lax.dot_general(
        onehot, stacked, (((1,), (1,)), ((), ())), precision=jax.lax.Precision.HIGHEST
    )
    hist_ref[...] += part

    @pl.when(pid == nsteps - 1)
    def _finish():
        h = hist_ref[...]
        cntf = h[:, 0:1]
        sc = h[:, 1:2]
        sa = h[:, 2:3]
        denom = jnp.maximum(cntf, 1.0)
        avg_conf = sc / denom
        avg_acc = sa / denom
        prop = cntf / jnp.float32(n_total)
        nonempty = cntf > 0.0
        ece_bins = jnp.where(nonempty, jnp.abs(avg_conf - avg_acc) * prop, 0.0)
        acc_bins = jnp.where(nonempty, avg_acc * prop, 0.0)
        ece_ref[...] = jnp.sum(ece_bins, axis=0, keepdims=True).reshape(1, 1) * 100.0
        acc_ref[...] = jnp.sum(acc_bins, axis=0, keepdims=True).reshape(1, 1) * 100.0


@jax.jit
def kernel(logits, labels):
    n, c = logits.shape
    block = 4096
    grid = n // block

    bounds = jnp.linspace(0.0, 1.0, _N_BINS + 1)
    lowers = jnp.concatenate([bounds[:-1], jnp.full((1,), 2.0, jnp.float32)])
    uppers = jnp.concatenate([bounds[1:], jnp.full((1,), 2.0, jnp.float32)])
    bounds2 = jnp.stack([lowers, uppers])

    labels3 = labels.astype(jnp.int32).reshape(grid, 1, block)

    ece, acc = pl.pallas_call(
        functools.partial(_ece_body, n),
        grid=(grid,),
        in_specs=[
            pl.BlockSpec((2, _LANES), lambda i: (0, 0)),
            pl.BlockSpec((block, c), lambda i: (i, 0)),
            pl.BlockSpec((1, 1, block), lambda i: (i, 0, 0)),
        ],
        out_specs=[
            pl.BlockSpec((1, 1), lambda i: (0, 0)),
            pl.BlockSpec((1, 1), lambda i: (0, 0)),
        ],
        out_shape=[
            jax.ShapeDtypeStruct((1, 1), jnp.float32),
            jax.ShapeDtypeStruct((1, 1), jnp.float32),
        ],
        scratch_shapes=[pltpu.VMEM((_LANES, 3), jnp.float32)],
        compiler_params=pltpu.CompilerParams(
            dimension_semantics=("arbitrary",),
        ),
    )(bounds2, logits, labels3)
    return ece.reshape(1), acc.reshape(1)


# speedup vs baseline: 1.7549x; 1.4692x over previous
import functools

import jax
import jax.numpy as jnp
from jax.experimental import pallas as pl
from jax.experimental.pallas import tpu as pltpu

_N_BINS = 15
_LANES = 16


def _ece_body(n_total, bounds_ref, logits_ref, labels_ref, ece_ref, acc_ref, hist_ref):
    pid = pl.program_id(0)
    nsteps = pl.num_programs(0)

    @pl.when(pid == 0)
    def _init():
        hist_ref[...] = jnp.zeros_like(hist_ref)

    x = logits_ref[...]
    b, c = x.shape
    xt = jax.lax.transpose(x, (1, 0))

    conf = jnp.max(xt, axis=0, keepdims=True)
    row = jax.lax.broadcasted_iota(jnp.int32, (c, b), 0)
    pred = jnp.min(
        jnp.where(xt == conf, row, jnp.int32(c)), axis=0, keepdims=True
    )
    lab = labels_ref[...].reshape(1, b)
    accv = (pred == lab).astype(jnp.float32)

    lowers = bounds_ref[0:1, :].reshape(_LANES, 1)
    uppers = bounds_ref[1:2, :].reshape(_LANES, 1)
    onehot = ((conf > lowers) & (conf <= uppers)).astype(jnp.float32)

    ones = jnp.ones((1, b), jnp.float32)
    stacked = jnp.concatenate([ones, conf, accv], axis=0)
    part = jax.lax.dot_general(
        onehot, stacked, (((1,), (1,)), ((), ())), precision=jax.lax.Precision.HIGHEST
    )
    hist_ref[...] += part

    @pl.when(pid == nsteps - 1)
    def _finish():
        h = hist_ref[...]
        cntf = h[:, 0:1]
        sc = h[:, 1:2]
        sa = h[:, 2:3]
        denom = jnp.maximum(cntf, 1.0)
        avg_conf = sc / denom
        avg_acc = sa / denom
        prop = cntf / jnp.float32(n_total)
        nonempty = cntf > 0.0
        ece_bins = jnp.where(nonempty, jnp.abs(avg_conf - avg_acc) * prop, 0.0)
        acc_bins = jnp.where(nonempty, avg_acc * prop, 0.0)
        ece_ref[...] = jnp.sum(ece_bins, axis=0, keepdims=True).reshape(1, 1) * 100.0
        acc_ref[...] = jnp.sum(acc_bins, axis=0, keepdims=True).reshape(1, 1) * 100.0


@jax.jit
def kernel(logits, labels):
    n, c = logits.shape
    block = 4096
    grid = n // block

    bounds = jnp.linspace(0.0, 1.0, _N_BINS + 1)
    lowers = jnp.concatenate([bounds[:-1], jnp.full((1,), 2.0, jnp.float32)])
    uppers = jnp.concatenate([bounds[1:], jnp.full((1,), 2.0, jnp.float32)])
    bounds2 = jnp.stack([lowers, uppers])

    labels3 = labels.astype(jnp.int32).reshape(grid, 1, block)

    ece, acc = pl.pallas_call(
        functools.partial(_ece_body, n),
        grid=(grid,),
        in_specs=[
            pl.BlockSpec((2, _LANES), lambda i: (0, 0)),
            pl.BlockSpec((block, c), lambda i: (i, 0)),
            pl.BlockSpec((1, 1, block), lambda i: (i, 0, 0)),
        ],
        out_specs=[
            pl.BlockSpec((1, 1), lambda i: (0, 0)),
            pl.BlockSpec((1, 1), lambda i: (0, 0)),
        ],
        out_shape=[
            jax.ShapeDtypeStruct((1, 1), jnp.float32),
            jax.ShapeDtypeStruct((1, 1), jnp.float32),
        ],
        scratch_shapes=[pltpu.VMEM((_LANES, 3), jnp.float32)],
        compiler_params=pltpu.CompilerParams(
            dimension_semantics=("arbitrary",),
        ),
    )(bounds2, logits, labels3)
    return ece.reshape(1), acc.reshape(1)
